# baseline (speedup 1.0000x reference)
import functools

import jax
import jax.numpy as jnp
from jax import lax
from jax.experimental import pallas as pl
from jax.experimental.pallas import tpu as pltpu
from jax.experimental.pallas import tpu_sc as plsc

N = 10000
E = 320000
D_IN = 128
D_H = 256
D_OUT = 64

NPAD = 10240
NC = 2
NS = 16
NW = NC * NS
CHUNK = 128
CH16 = 160
ESUB = CH16 * CHUNK
EPAD16 = NS * ESUB
ROWS_PER_SUB = NPAD // NS

_mesh = plsc.VectorSubcoreMesh(
    core_axis_name="c", subcore_axis_name="s", num_cores=NC, num_subcores=NS)
_sc_params = pltpu.CompilerParams(
    needs_layout_passes=False, use_tc_tiling_on_sc=False)


@functools.partial(
    pl.kernel,
    out_type=[jax.ShapeDtypeStruct((1, NPAD), jnp.float32),
              jax.ShapeDtypeStruct((NS, ESUB), jnp.float32)],
    mesh=_mesh,
    compiler_params=_sc_params,
    scratch_types=[
        pltpu.VMEM((ESUB,), jnp.int32),
        pltpu.VMEM((ESUB,), jnp.int32),
        pltpu.VMEM((ESUB,), jnp.float32),
        pltpu.VMEM((NPAD,), jnp.float32),
        pltpu.VMEM((NS, ROWS_PER_SUB), jnp.float32),
        pltpu.VMEM((NPAD,), jnp.float32),
        pltpu.VMEM_SHARED((NS, NPAD), jnp.float32),
        pltpu.VMEM_SHARED((NPAD,), jnp.float32),
    ],
)
def _norm_kernel(src_hbm, dst_hbm, ew_hbm, dis_hbm, warr_hbm,
                 srcb, dstb, ewb, degl, degbuf, disl, degs_sh, dis_sh):
    cid = lax.axis_index("c")
    sid = lax.axis_index("s")
    rbase = sid * ROWS_PER_SUB
    pltpu.sync_copy(src_hbm.at[sid], srcb)
    pltpu.sync_copy(dst_hbm.at[sid], dstb)
    pltpu.sync_copy(ew_hbm.at[sid], ewb)

    def zero(i, _):
        degl[pl.ds(i * 16, 16)] = jnp.zeros((16,), jnp.float32)
        return 0

    lax.fori_loop(0, NPAD // 16, zero, 0)

    def hist(t, _):
        sl = pl.ds(t * 16, 16)
        plsc.addupdate_scatter(degl, [dstb[sl]], ewb[sl])
        return 0

    lax.fori_loop(0, ESUB // 16, hist, 0)
    pltpu.sync_copy(degl, degs_sh.at[sid])
    plsc.subcore_barrier()

    def fetch(r, _):
        pltpu.sync_copy(degs_sh.at[r, pl.ds(rbase, ROWS_PER_SUB)],
                        degbuf.at[r])
        return 0

    lax.fori_loop(0, NS, fetch, 0)
    magic = jnp.full((16,), 0x5F3759DF, jnp.int32)

    def red(t, _):
        s = degbuf[0, pl.ds(t * 16, 16)]
        for r in range(1, NS):
            s = s + degbuf[r, pl.ds(t * 16, 16)]
        d = s + 1.0
        i = plsc.bitcast(d, jnp.int32)
        y = plsc.bitcast(magic - (i >> 1), jnp.float32)
        for _ in range(3):
            y = y * (1.5 - 0.5 * d * y * y)
        disl[pl.ds(rbase + t * 16, 16)] = y
        return 0

    lax.fori_loop(0, ROWS_PER_SUB // 16, red, 0)
    pltpu.sync_copy(disl.at[pl.ds(rbase, ROWS_PER_SUB)],
                    dis_sh.at[pl.ds(rbase, ROWS_PER_SUB)])
    plsc.subcore_barrier()
    pltpu.sync_copy(dis_sh, disl)

    def mkw(t, _):
        sl = pl.ds(t * 16, 16)
        sv = plsc.load_gather(disl, [srcb[sl]])
        dv = plsc.load_gather(disl, [dstb[sl]])
        ewb[sl] = ewb[sl] * sv * dv
        return 0

    lax.fori_loop(0, ESUB // 16, mkw, 0)

    @pl.when(cid == 0)
    def _():
        pltpu.sync_copy(ewb, warr_hbm.at[sid])
        pltpu.sync_copy(dis_sh.at[pl.ds(rbase, ROWS_PER_SUB)],
                        dis_hbm.at[0, pl.ds(rbase, ROWS_PER_SUB)])


def _make_agg_kernel(D2, bf16_table):
    tdt = jnp.bfloat16 if bf16_table else jnp.float32

    @functools.partial(
        pl.kernel,
        out_type=jax.ShapeDtypeStruct((NC, NPAD, D2), jnp.float32),
        mesh=_mesh,
        compiler_params=_sc_params,
        scratch_types=[
            pltpu.VMEM((CH16, CHUNK), jnp.int32),
            pltpu.VMEM((CH16, CHUNK), jnp.int32),
            pltpu.VMEM((ESUB,), jnp.float32),
            pltpu.VMEM((CHUNK, D2), tdt),
            pltpu.VMEM((CHUNK, D2), tdt),
            pltpu.VMEM((CHUNK, D2), tdt),
            pltpu.VMEM((CHUNK, D2), jnp.float32),
            pltpu.VMEM((CHUNK, D2), jnp.float32),
            pltpu.VMEM((CHUNK, D2), jnp.float32),
            pltpu.VMEM_SHARED((NPAD, D2), jnp.float32),
            pltpu.SemaphoreType.DMA,
            pltpu.SemaphoreType.DMA,
            pltpu.SemaphoreType.DMA,
            pltpu.SemaphoreType.DMA,
            pltpu.SemaphoreType.DMA,
            pltpu.SemaphoreType.DMA,
        ],
    )
    def agg(table_hbm, src_hbm, dst_hbm, w_hbm, zeros_hbm, out_hbm,
            srcb, dstb, ewb, r0, r1, r2, f0, f1, f2, acc,
            g0, g1, g2, s0, s1, s2):
        cid = lax.axis_index("c")
        sid = lax.axis_index("s")
        rbase = sid * ROWS_PER_SUB
        rows = (r0, r1, r2)
        frows = (f0, f1, f2) if bf16_table else (r0, r1, r2)
        gsems = (g0, g1, g2)
        ssems = (s0, s1, s2)
        pltpu.sync_copy(zeros_hbm, acc.at[pl.ds(rbase, ROWS_PER_SUB)])
        pltpu.sync_copy(src_hbm.at[sid], srcb)
        pltpu.sync_copy(dst_hbm.at[sid], dstb)
        pltpu.sync_copy(w_hbm.at[sid], ewb)
        plsc.subcore_barrier()

        def start_g(ci, b):
            pltpu.async_copy(table_hbm.at[cid].at[srcb.at[ci]],
                             rows[b], gsems[b])

        def wait_g(ci, b):
            pltpu.make_async_copy(table_hbm.at[cid].at[srcb.at[ci]],
                                  rows[b], gsems[b]).wait()

        def compute(ci, b):
            buf = rows[b]
            fbuf = frows[b]

            nj = D2 // 16

            def scale(g, _):
                base = g * 16
                ewv = ewb[pl.ds(ci * CHUNK + base, 16)]

                def sub(lg, _):
                    lbase = base + lg * 4
                    wvs = [
                        ewv.at[jnp.full((16,), lg * 4 + k, jnp.int32)].get(
                            mode="promise_in_bounds")
                        for k in range(4)
                    ]
                    if bf16_table:
                        vals = [
                            [v
                             for gg in range(D2 // 32)
                             for v in plsc.unpack(
                                 buf[lbase + k, pl.ds(gg * 32, 32)],
                                 format=plsc.PackFormat.INTERLEAVED,
                                 preferred_element_type=jnp.float32)]
                            for k in range(4)
                        ]
                    else:
                        vals = [
                            [buf[lbase + k, pl.ds(j * 16, 16)]
                             for j in range(nj)]
                            for k in range(4)
                        ]
                    for k in range(4):
                        for j in range(nj):
                            fbuf[lbase + k, pl.ds(j * 16, 16)] = (
                                vals[k][j] * wvs[k])
                    return 0

                lax.fori_loop(0, 4, sub, 0)
                return 0

            lax.fori_loop(0, CHUNK // 16, scale, 0)

        def start_s(ci, b):
            pltpu.async_copy(frows[b], acc.at[dstb.at[ci]], ssems[b],
                             add=True)

        def wait_s(ci, b):
            pltpu.make_async_copy(frows[b], acc.at[dstb.at[ci]],
                                  ssems[b]).wait()

        def phase(ci, b, do_wait_s, do_start_g):
            wait_g(ci, b)
            compute(ci, b)
            if do_wait_s:
                wait_s(ci - 1, (b + 2) % 3)
            start_s(ci, b)
            if do_start_g:
                start_g(ci + 2, (b + 2) % 3)

        start_g(0, 0)
        start_g(1, 1)
        phase(0, 0, False, True)
        phase(1, 1, True, True)

        def body(g, _):
            phase(3 * g + 2, 2, True, True)
            phase(3 * g + 3, 0, True, True)
            phase(3 * g + 4, 1, True, True)
            return 0

        lax.fori_loop(0, (CH16 - 4) // 3, body, 0)
        phase(CH16 - 2, 2, True, False)
        phase(CH16 - 1, 0, True, False)
        wait_s(CH16 - 1, 0)
        plsc.subcore_barrier()
        pltpu.sync_copy(acc.at[pl.ds(rbase, ROWS_PER_SUB)],
                        out_hbm.at[cid, pl.ds(rbase, ROWS_PER_SUB)])

    return agg


_agg1 = _make_agg_kernel(D_IN // 2, bf16_table=False)
_agg2 = _make_agg_kernel(D_OUT // 2, bf16_table=False)

_PERM = [h * 64 + g * 32 + (2 * k if k < 16 else 2 * (k - 16) + 1)
         for h in range(2) for g in range(2) for k in range(32)]


RB = 1024
NBB = NPAD // RB


H_IN = D_IN // 2
H_OUT = D_OUT // 2


def _dis_col2(dis_row):
    r = lax.broadcasted_iota(jnp.int32, (RB, RB), 0)
    c = lax.broadcasted_iota(jnp.int32, (RB, RB), 1)
    dis_col = jnp.sum(
        jnp.where(r == c, jnp.broadcast_to(dis_row, (RB, RB)), 0.0),
        axis=1, keepdims=True)
    return dis_col * dis_col


def _mid_body(acc_ref, x_ref, dis_ref, w1_ref, b1_ref, w2_ref, zs_ref):
    dis2 = _dis_col2(dis_ref[...])
    t = jnp.concatenate([acc_ref[0], acc_ref[1]], axis=1) \
        + x_ref[...] * dis2
    h = jnp.maximum(
        jnp.dot(t, w1_ref[...], preferred_element_type=jnp.float32)
        + b1_ref[...], 0.0)
    z = jnp.dot(h, w2_ref[...], preferred_element_type=jnp.float32)
    zs_ref[0] = z[:, :H_OUT]
    zs_ref[1] = z[:, H_OUT:]


def _fin_body(acc_ref, zs_ref, dis_ref, b2_ref, out_ref):
    dis2 = _dis_col2(dis_ref[...])
    z = jnp.concatenate([zs_ref[0], zs_ref[1]], axis=1)
    t = jnp.concatenate([acc_ref[0], acc_ref[1]], axis=1)
    out_ref[...] = t + z * dis2 + b2_ref[...]


_mid_call = pl.pallas_call(
    _mid_body,
    grid=(NBB,),
    in_specs=[
        pl.BlockSpec((NC, RB, H_IN), lambda r: (0, r, 0)),
        pl.BlockSpec((RB, D_IN), lambda r: (r, 0)),
        pl.BlockSpec((1, RB), lambda r: (0, r)),
        pl.BlockSpec((D_IN, D_H), lambda r: (0, 0)),
        pl.BlockSpec((1, D_H), lambda r: (0, 0)),
        pl.BlockSpec((D_H, D_OUT), lambda r: (0, 0)),
    ],
    out_specs=pl.BlockSpec((NC, RB, H_OUT), lambda r: (0, r, 0)),
    out_shape=jax.ShapeDtypeStruct((NC, NPAD, H_OUT), jnp.float32),
)

_fin_call = pl.pallas_call(
    _fin_body,
    grid=(NBB,),
    in_specs=[
        pl.BlockSpec((NC, RB, H_OUT), lambda r: (0, r, 0)),
        pl.BlockSpec((NC, RB, H_OUT), lambda r: (0, r, 0)),
        pl.BlockSpec((1, RB), lambda r: (0, r)),
        pl.BlockSpec((1, D_OUT), lambda r: (0, 0)),
    ],
    out_specs=pl.BlockSpec((RB, D_OUT), lambda r: (r, 0)),
    out_shape=jax.ShapeDtypeStruct((NPAD, D_OUT), jnp.float32),
)


def kernel(x, edge_index, edge_weight, W1, b1, W2, b2):
    src = edge_index[0].astype(jnp.int32)
    dst = edge_index[1].astype(jnp.int32)
    ew = edge_weight.astype(jnp.float32)

    pad16 = EPAD16 - E
    src3 = jnp.concatenate(
        [src, jnp.zeros((pad16,), jnp.int32)]).reshape(NS, CH16, CHUNK)
    dst3 = jnp.concatenate(
        [dst, jnp.zeros((pad16,), jnp.int32)]).reshape(NS, CH16, CHUNK)
    ew16 = jnp.concatenate(
        [ew, jnp.zeros((pad16,), jnp.float32)]).reshape(NS, ESUB)

    x_p = jnp.concatenate([x, jnp.zeros((NPAD - N, D_IN), jnp.float32)])
    xsplit = jnp.stack([x_p[:, :H_IN], x_p[:, H_IN:]])
    zeros_h1 = jnp.zeros((ROWS_PER_SUB, H_IN), jnp.float32)
    zeros_h2 = jnp.zeros((ROWS_PER_SUB, H_OUT), jnp.float32)

    dis, warr = _norm_kernel(src3.reshape(NS, ESUB),
                             dst3.reshape(NS, ESUB), ew16)
    acc1 = _agg1(xsplit, src3, dst3, warr, zeros_h1)
    zss = _mid_call(acc1, x_p, dis, W1, b1.reshape(1, D_H), W2)

    acc2 = _agg2(zss, src3, dst3, warr, zeros_h2)
    out = _fin_call(acc2, zss, dis, b2.reshape(1, D_OUT))
    return out[:N]


# speedup vs baseline: 1.1265x; 1.1265x over previous
import functools

import jax
import jax.numpy as jnp
from jax import lax
from jax.experimental import pallas as pl
from jax.experimental.pallas import tpu as pltpu
from jax.experimental.pallas import tpu_sc as plsc

N = 10000
E = 320000
D_IN = 128
D_H = 256
D_OUT = 64

NPAD = 10240
NC = 2
NS = 16
NW = NC * NS
CHUNK = 128
CH16 = 3 * -(-E // (NS * CHUNK * 3))
ESUB = CH16 * CHUNK
EPAD16 = NS * ESUB
ROWS_PER_SUB = NPAD // NS

_mesh = plsc.VectorSubcoreMesh(
    core_axis_name="c", subcore_axis_name="s", num_cores=NC, num_subcores=NS)
_sc_params = pltpu.CompilerParams(
    needs_layout_passes=False, use_tc_tiling_on_sc=False)


@functools.partial(
    pl.kernel,
    out_type=[jax.ShapeDtypeStruct((1, NPAD), jnp.float32),
              jax.ShapeDtypeStruct((NS, ESUB), jnp.float32)],
    mesh=_mesh,
    compiler_params=_sc_params,
    scratch_types=[
        pltpu.VMEM((ESUB,), jnp.int32),
        pltpu.VMEM((ESUB,), jnp.int32),
        pltpu.VMEM((ESUB,), jnp.float32),
        pltpu.VMEM((NPAD,), jnp.float32),
        pltpu.VMEM((NS, ROWS_PER_SUB), jnp.float32),
        pltpu.VMEM((NPAD,), jnp.float32),
        pltpu.VMEM_SHARED((NS, NPAD), jnp.float32),
        pltpu.VMEM_SHARED((NPAD,), jnp.float32),
    ],
)
def _norm_kernel(src_hbm, dst_hbm, ew_hbm, dis_hbm, warr_hbm,
                 srcb, dstb, ewb, degl, degbuf, disl, degs_sh, dis_sh):
    cid = lax.axis_index("c")
    sid = lax.axis_index("s")
    rbase = sid * ROWS_PER_SUB
    pltpu.sync_copy(src_hbm.at[sid], srcb)
    pltpu.sync_copy(dst_hbm.at[sid], dstb)
    pltpu.sync_copy(ew_hbm.at[sid], ewb)

    def zero(i, _):
        degl[pl.ds(i * 16, 16)] = jnp.zeros((16,), jnp.float32)
        return 0

    lax.fori_loop(0, NPAD // 16, zero, 0)

    def hist(t, _):
        sl = pl.ds(t * 16, 16)
        plsc.addupdate_scatter(degl, [dstb[sl]], ewb[sl])
        return 0

    lax.fori_loop(0, ESUB // 16, hist, 0)
    pltpu.sync_copy(degl, degs_sh.at[sid])
    plsc.subcore_barrier()

    def fetch(r, _):
        pltpu.sync_copy(degs_sh.at[r, pl.ds(rbase, ROWS_PER_SUB)],
                        degbuf.at[r])
        return 0

    lax.fori_loop(0, NS, fetch, 0)
    magic = jnp.full((16,), 0x5F3759DF, jnp.int32)

    def red(t, _):
        s = degbuf[0, pl.ds(t * 16, 16)]
        for r in range(1, NS):
            s = s + degbuf[r, pl.ds(t * 16, 16)]
        d = s + 1.0
        i = plsc.bitcast(d, jnp.int32)
        y = plsc.bitcast(magic - (i >> 1), jnp.float32)
        for _ in range(3):
            y = y * (1.5 - 0.5 * d * y * y)
        disl[pl.ds(rbase + t * 16, 16)] = y
        return 0

    lax.fori_loop(0, ROWS_PER_SUB // 16, red, 0)
    pltpu.sync_copy(disl.at[pl.ds(rbase, ROWS_PER_SUB)],
                    dis_sh.at[pl.ds(rbase, ROWS_PER_SUB)])
    plsc.subcore_barrier()
    pltpu.sync_copy(dis_sh, disl)

    def mkw(t, _):
        sl = pl.ds(t * 16, 16)
        sv = plsc.load_gather(disl, [srcb[sl]])
        dv = plsc.load_gather(disl, [dstb[sl]])
        ewb[sl] = ewb[sl] * sv * dv
        return 0

    lax.fori_loop(0, ESUB // 16, mkw, 0)

    @pl.when(cid == 0)
    def _():
        pltpu.sync_copy(ewb, warr_hbm.at[sid])
        pltpu.sync_copy(dis_sh.at[pl.ds(rbase, ROWS_PER_SUB)],
                        dis_hbm.at[0, pl.ds(rbase, ROWS_PER_SUB)])


def _make_agg_kernel(D2):
    @functools.partial(
        pl.kernel,
        out_type=jax.ShapeDtypeStruct((NC, NPAD, D2), jnp.float32),
        mesh=_mesh,
        compiler_params=_sc_params,
        scratch_types=[
            pltpu.VMEM((CH16, CHUNK), jnp.int32),
            pltpu.VMEM((CH16, CHUNK), jnp.int32),
            pltpu.VMEM((ESUB,), jnp.float32),
            pltpu.VMEM((CHUNK, D2), jnp.float32),
            pltpu.VMEM((CHUNK, D2), jnp.float32),
            pltpu.VMEM((CHUNK, D2), jnp.float32),
            pltpu.VMEM_SHARED((NPAD, D2), jnp.float32),
            pltpu.SemaphoreType.DMA,
            pltpu.SemaphoreType.DMA,
            pltpu.SemaphoreType.DMA,
            pltpu.SemaphoreType.DMA,
            pltpu.SemaphoreType.DMA,
            pltpu.SemaphoreType.DMA,
        ],
    )
    def agg(table_hbm, src_hbm, dst_hbm, w_hbm, zeros_hbm, out_hbm,
            srcb, dstb, ewb, r0, r1, r2, acc, g0, g1, g2, s0, s1, s2):
        cid = lax.axis_index("c")
        sid = lax.axis_index("s")
        rbase = sid * ROWS_PER_SUB
        rows = (r0, r1, r2)
        gsems = (g0, g1, g2)
        ssems = (s0, s1, s2)
        pltpu.sync_copy(zeros_hbm, acc.at[pl.ds(rbase, ROWS_PER_SUB)])
        pltpu.sync_copy(src_hbm.at[sid], srcb)
        pltpu.sync_copy(dst_hbm.at[sid], dstb)
        pltpu.sync_copy(w_hbm.at[sid], ewb)
        plsc.subcore_barrier()

        def start_g(ci, b):
            pltpu.async_copy(table_hbm.at[cid].at[srcb.at[ci]],
                             rows[b], gsems[b])

        def wait_g(ci, b):
            pltpu.make_async_copy(table_hbm.at[cid].at[srcb.at[ci]],
                                  rows[b], gsems[b]).wait()

        def compute(ci, b):
            buf = rows[b]

            nj = D2 // 16

            def scale(g, _):
                base = g * 16
                ewv = ewb[pl.ds(ci * CHUNK + base, 16)]

                def sub(lg, _):
                    lbase = base + lg * 4
                    wvs = [
                        ewv.at[jnp.full((16,), lg * 4 + k, jnp.int32)].get(
                            mode="promise_in_bounds")
                        for k in range(4)
                    ]
                    vals = [
                        [buf[lbase + k, pl.ds(j * 16, 16)]
                         for j in range(nj)]
                        for k in range(4)
                    ]
                    for k in range(4):
                        for j in range(nj):
                            buf[lbase + k, pl.ds(j * 16, 16)] = (
                                vals[k][j] * wvs[k])
                    return 0

                lax.fori_loop(0, 4, sub, 0)
                return 0

            lax.fori_loop(0, CHUNK // 16, scale, 0)

        def start_s(ci, b):
            pltpu.async_copy(rows[b], acc.at[dstb.at[ci]], ssems[b],
                             add=True)

        def wait_s(ci, b):
            pltpu.make_async_copy(rows[b], acc.at[dstb.at[ci]],
                                  ssems[b]).wait()

        def phase(ci, b, do_wait_s, do_start_g):
            wait_g(ci, b)
            compute(ci, b)
            if do_wait_s:
                wait_s(ci - 1, (b + 2) % 3)
            start_s(ci, b)
            if do_start_g:
                start_g(ci + 2, (b + 2) % 3)

        start_g(0, 0)
        start_g(1, 1)
        phase(0, 0, False, True)
        phase(1, 1, True, True)

        def body(g, _):
            phase(3 * g + 2, 2, True, True)
            phase(3 * g + 3, 0, True, True)
            phase(3 * g + 4, 1, True, True)
            return 0

        lax.fori_loop(0, (CH16 - 8) // 3 + 1, body, 0)
        phase(CH16 - 4, 2, True, True)
        phase(CH16 - 3, 0, True, True)
        phase(CH16 - 2, 1, True, False)
        phase(CH16 - 1, 2, True, False)
        wait_s(CH16 - 1, 2)
        plsc.subcore_barrier()
        pltpu.sync_copy(acc.at[pl.ds(rbase, ROWS_PER_SUB)],
                        out_hbm.at[cid, pl.ds(rbase, ROWS_PER_SUB)])

    return agg


_agg1 = _make_agg_kernel(D_IN // 2)
_agg2 = _make_agg_kernel(D_OUT // 2)


RB = 1024
NBB = NPAD // RB


H_IN = D_IN // 2
H_OUT = D_OUT // 2


def _dis_col2(dis_row):
    r = lax.broadcasted_iota(jnp.int32, (RB, RB), 0)
    c = lax.broadcasted_iota(jnp.int32, (RB, RB), 1)
    dis_col = jnp.sum(
        jnp.where(r == c, jnp.broadcast_to(dis_row, (RB, RB)), 0.0),
        axis=1, keepdims=True)
    return dis_col * dis_col


def _mid_body(acc_ref, x_ref, dis_ref, w1_ref, b1_ref, w2_ref, zs_ref):
    dis2 = _dis_col2(dis_ref[...])
    t = jnp.concatenate([acc_ref[0], acc_ref[1]], axis=1) \
        + x_ref[...] * dis2
    h = jnp.maximum(
        jnp.dot(t, w1_ref[...], preferred_element_type=jnp.float32)
        + b1_ref[...], 0.0)
    z = jnp.dot(h, w2_ref[...], preferred_element_type=jnp.float32)
    zs_ref[0] = z[:, :H_OUT]
    zs_ref[1] = z[:, H_OUT:]


def _fin_body(acc_ref, zs_ref, dis_ref, b2_ref, out_ref):
    dis2 = _dis_col2(dis_ref[...])
    z = jnp.concatenate([zs_ref[0], zs_ref[1]], axis=1)
    t = jnp.concatenate([acc_ref[0], acc_ref[1]], axis=1)
    out_ref[...] = t + z * dis2 + b2_ref[...]


_mid_call = pl.pallas_call(
    _mid_body,
    grid=(NBB,),
    in_specs=[
        pl.BlockSpec((NC, RB, H_IN), lambda r: (0, r, 0)),
        pl.BlockSpec((RB, D_IN), lambda r: (r, 0)),
        pl.BlockSpec((1, RB), lambda r: (0, r)),
        pl.BlockSpec((D_IN, D_H), lambda r: (0, 0)),
        pl.BlockSpec((1, D_H), lambda r: (0, 0)),
        pl.BlockSpec((D_H, D_OUT), lambda r: (0, 0)),
    ],
    out_specs=pl.BlockSpec((NC, RB, H_OUT), lambda r: (0, r, 0)),
    out_shape=jax.ShapeDtypeStruct((NC, NPAD, H_OUT), jnp.float32),
)

_fin_call = pl.pallas_call(
    _fin_body,
    grid=(NBB,),
    in_specs=[
        pl.BlockSpec((NC, RB, H_OUT), lambda r: (0, r, 0)),
        pl.BlockSpec((NC, RB, H_OUT), lambda r: (0, r, 0)),
        pl.BlockSpec((1, RB), lambda r: (0, r)),
        pl.BlockSpec((1, D_OUT), lambda r: (0, 0)),
    ],
    out_specs=pl.BlockSpec((RB, D_OUT), lambda r: (r, 0)),
    out_shape=jax.ShapeDtypeStruct((NPAD, D_OUT), jnp.float32),
)


def kernel(x, edge_index, edge_weight, W1, b1, W2, b2):
    src = edge_index[0].astype(jnp.int32)
    dst = edge_index[1].astype(jnp.int32)
    ew = edge_weight.astype(jnp.float32)

    pad16 = EPAD16 - E
    src3 = jnp.concatenate(
        [src, jnp.zeros((pad16,), jnp.int32)]).reshape(NS, CH16, CHUNK)
    dst3 = jnp.concatenate(
        [dst, jnp.zeros((pad16,), jnp.int32)]).reshape(NS, CH16, CHUNK)
    ew16 = jnp.concatenate(
        [ew, jnp.zeros((pad16,), jnp.float32)]).reshape(NS, ESUB)

    x_p = jnp.concatenate([x, jnp.zeros((NPAD - N, D_IN), jnp.float32)])
    xsplit = jnp.stack([x_p[:, :H_IN], x_p[:, H_IN:]])
    zeros_h1 = jnp.zeros((ROWS_PER_SUB, H_IN), jnp.float32)
    zeros_h2 = jnp.zeros((ROWS_PER_SUB, H_OUT), jnp.float32)

    dis, warr = _norm_kernel(src3.reshape(NS, ESUB),
                             dst3.reshape(NS, ESUB), ew16)
    acc1 = _agg1(xsplit, src3, dst3, warr, zeros_h1)
    zss = _mid_call(acc1, x_p, dis, W1, b1.reshape(1, D_H), W2)

    acc2 = _agg2(zss, src3, dst3, warr, zeros_h2)
    out = _fin_call(acc2, zss, dis, b2.reshape(1, D_OUT))
    return out[:N]


# speedup vs baseline: 1.1270x; 1.0005x over previous
import functools

import jax
import jax.numpy as jnp
from jax import lax
from jax.experimental import pallas as pl
from jax.experimental.pallas import tpu as pltpu
from jax.experimental.pallas import tpu_sc as plsc

N = 10000
E = 320000
D_IN = 128
D_H = 256
D_OUT = 64

NPAD = 10240
NC = 2
NS = 16
CHUNK = 128
CH16 = 3 * -(-E // (NS * CHUNK * 3))
ESUB = CH16 * CHUNK
EPAD16 = NS * ESUB
ROWS_PER_SUB = NPAD // NS

_mesh = plsc.VectorSubcoreMesh(
    core_axis_name="c", subcore_axis_name="s", num_cores=NC, num_subcores=NS)
_sc_params = pltpu.CompilerParams(
    needs_layout_passes=False, use_tc_tiling_on_sc=False)


@functools.partial(
    pl.kernel,
    out_type=[jax.ShapeDtypeStruct((1, NPAD), jnp.float32),
              jax.ShapeDtypeStruct((NS, ESUB), jnp.float32)],
    mesh=_mesh,
    compiler_params=_sc_params,
    scratch_types=[
        pltpu.VMEM((ESUB,), jnp.int32),
        pltpu.VMEM((ESUB,), jnp.int32),
        pltpu.VMEM((ESUB,), jnp.float32),
        pltpu.VMEM((NPAD,), jnp.float32),
        pltpu.VMEM((NS, ROWS_PER_SUB), jnp.float32),
        pltpu.VMEM((NPAD,), jnp.float32),
        pltpu.VMEM_SHARED((NS, NPAD), jnp.float32),
        pltpu.VMEM_SHARED((NPAD,), jnp.float32),
    ],
)
def _norm_kernel(src_hbm, dst_hbm, ew_hbm, dis_hbm, warr_hbm,
                 srcb, dstb, ewb, degl, degbuf, disl, degs_sh, dis_sh):
    cid = lax.axis_index("c")
    sid = lax.axis_index("s")
    rbase = sid * ROWS_PER_SUB
    pltpu.sync_copy(src_hbm.at[sid], srcb)
    pltpu.sync_copy(dst_hbm.at[sid], dstb)
    pltpu.sync_copy(ew_hbm.at[sid], ewb)

    def zero(i, _):
        degl[pl.ds(i * 16, 16)] = jnp.zeros((16,), jnp.float32)
        return 0

    lax.fori_loop(0, NPAD // 16, zero, 0)

    def hist(t, _):
        sl = pl.ds(t * 16, 16)
        plsc.addupdate_scatter(degl, [dstb[sl]], ewb[sl])
        return 0

    lax.fori_loop(0, ESUB // 16, hist, 0)
    pltpu.sync_copy(degl, degs_sh.at[sid])
    plsc.subcore_barrier()

    def fetch(r, _):
        pltpu.sync_copy(degs_sh.at[r, pl.ds(rbase, ROWS_PER_SUB)],
                        degbuf.at[r])
        return 0

    lax.fori_loop(0, NS, fetch, 0)
    magic = jnp.full((16,), 0x5F3759DF, jnp.int32)

    def red(t, _):
        s = degbuf[0, pl.ds(t * 16, 16)]
        for r in range(1, NS):
            s = s + degbuf[r, pl.ds(t * 16, 16)]
        d = s + 1.0
        i = plsc.bitcast(d, jnp.int32)
        y = plsc.bitcast(magic - (i >> 1), jnp.float32)
        for _ in range(3):
            y = y * (1.5 - 0.5 * d * y * y)
        disl[pl.ds(rbase + t * 16, 16)] = y
        return 0

    lax.fori_loop(0, ROWS_PER_SUB // 16, red, 0)
    pltpu.sync_copy(disl.at[pl.ds(rbase, ROWS_PER_SUB)],
                    dis_sh.at[pl.ds(rbase, ROWS_PER_SUB)])
    plsc.subcore_barrier()
    pltpu.sync_copy(dis_sh, disl)

    def mkw(t, _):
        sl = pl.ds(t * 16, 16)
        sv = plsc.load_gather(disl, [srcb[sl]])
        dv = plsc.load_gather(disl, [dstb[sl]])
        ewb[sl] = ewb[sl] * sv * dv
        return 0

    lax.fori_loop(0, ESUB // 16, mkw, 0)

    @pl.when(cid == 0)
    def _():
        pltpu.sync_copy(ewb, warr_hbm.at[sid])
        pltpu.sync_copy(dis_sh.at[pl.ds(rbase, ROWS_PER_SUB)],
                        dis_hbm.at[0, pl.ds(rbase, ROWS_PER_SUB)])


def _make_agg_kernel(D2):
    @functools.partial(
        pl.kernel,
        out_type=jax.ShapeDtypeStruct((NC, NPAD, D2), jnp.float32),
        mesh=_mesh,
        compiler_params=_sc_params,
        scratch_types=[
            pltpu.VMEM((CH16, CHUNK), jnp.int32),
            pltpu.VMEM((CH16, CHUNK), jnp.int32),
            pltpu.VMEM((ESUB,), jnp.float32),
            pltpu.VMEM((CHUNK, D2), jnp.float32),
            pltpu.VMEM((CHUNK, D2), jnp.float32),
            pltpu.VMEM((CHUNK, D2), jnp.float32),
            pltpu.VMEM_SHARED((NPAD, D2), jnp.float32),
            pltpu.SemaphoreType.DMA,
            pltpu.SemaphoreType.DMA,
            pltpu.SemaphoreType.DMA,
            pltpu.SemaphoreType.DMA,
            pltpu.SemaphoreType.DMA,
            pltpu.SemaphoreType.DMA,
        ],
    )
    def agg(table_hbm, src_hbm, dst_hbm, w_hbm, zeros_hbm, out_hbm,
            srcb, dstb, ewb, r0, r1, r2, acc, g0, g1, g2, s0, s1, s2):
        cid = lax.axis_index("c")
        sid = lax.axis_index("s")
        rbase = sid * ROWS_PER_SUB
        rows = (r0, r1, r2)
        gsems = (g0, g1, g2)
        ssems = (s0, s1, s2)
        pltpu.sync_copy(zeros_hbm, acc.at[pl.ds(rbase, ROWS_PER_SUB)])
        pltpu.sync_copy(src_hbm.at[sid], srcb)
        pltpu.sync_copy(dst_hbm.at[sid], dstb)
        pltpu.sync_copy(w_hbm.at[sid], ewb)
        plsc.subcore_barrier()

        def start_g(ci, b):
            pltpu.async_copy(table_hbm.at[cid].at[srcb.at[ci]],
                             rows[b], gsems[b])

        def wait_g(ci, b):
            pltpu.make_async_copy(table_hbm.at[cid].at[srcb.at[ci]],
                                  rows[b], gsems[b]).wait()

        def compute(ci, b):
            buf = rows[b]

            nj = D2 // 16

            def scale(g, _):
                base = g * 16
                ewv = ewb[pl.ds(ci * CHUNK + base, 16)]

                def sub(lg, _):
                    lbase = base + lg * 4
                    wvs = [
                        ewv.at[jnp.full((16,), lg * 4 + k, jnp.int32)].get(
                            mode="promise_in_bounds")
                        for k in range(4)
                    ]
                    vals = [
                        [buf[lbase + k, pl.ds(j * 16, 16)]
                         for j in range(nj)]
                        for k in range(4)
                    ]
                    for k in range(4):
                        for j in range(nj):
                            buf[lbase + k, pl.ds(j * 16, 16)] = (
                                vals[k][j] * wvs[k])
                    return 0

                lax.fori_loop(0, 4, sub, 0)
                return 0

            lax.fori_loop(0, CHUNK // 16, scale, 0)

        def start_s(ci, b):
            pltpu.async_copy(rows[b], acc.at[dstb.at[ci]], ssems[b],
                             add=True)

        def wait_s(ci, b):
            pltpu.make_async_copy(rows[b], acc.at[dstb.at[ci]],
                                  ssems[b]).wait()

        def phase(ci, b, do_wait_s, do_start_g):
            wait_g(ci, b)
            compute(ci, b)
            if do_wait_s:
                wait_s(ci - 1, (b + 2) % 3)
            start_s(ci, b)
            if do_start_g:
                start_g(ci + 2, (b + 2) % 3)

        start_g(0, 0)
        start_g(1, 1)
        phase(0, 0, False, True)
        phase(1, 1, True, True)

        def body(g, _):
            phase(3 * g + 2, 2, True, True)
            phase(3 * g + 3, 0, True, True)
            phase(3 * g + 4, 1, True, True)
            return 0

        lax.fori_loop(0, (CH16 - 8) // 3 + 1, body, 0)
        phase(CH16 - 4, 2, True, True)
        phase(CH16 - 3, 0, True, True)
        phase(CH16 - 2, 1, True, False)
        phase(CH16 - 1, 2, True, False)
        wait_s(CH16 - 1, 2)
        plsc.subcore_barrier()
        pltpu.sync_copy(acc.at[pl.ds(rbase, ROWS_PER_SUB)],
                        out_hbm.at[cid, pl.ds(rbase, ROWS_PER_SUB)])

    return agg


_agg1 = _make_agg_kernel(D_IN // 2)
_agg2 = _make_agg_kernel(D_OUT // 2)


RB = 1024
NBB = NPAD // RB


H_IN = D_IN // 2
H_OUT = D_OUT // 2


def _dis_col2(dis_row):
    r = lax.broadcasted_iota(jnp.int32, (RB, RB), 0)
    c = lax.broadcasted_iota(jnp.int32, (RB, RB), 1)
    dis_col = jnp.sum(
        jnp.where(r == c, jnp.broadcast_to(dis_row, (RB, RB)), 0.0),
        axis=1, keepdims=True)
    return dis_col * dis_col


def _mid_body(acc_ref, x_ref, dis_ref, w1_ref, b1_ref, w2_ref, zs_ref):
    dis2 = _dis_col2(dis_ref[...])
    t = jnp.concatenate([acc_ref[0], acc_ref[1]], axis=1) \
        + x_ref[...] * dis2
    h = jnp.maximum(
        jnp.dot(t, w1_ref[...], preferred_element_type=jnp.float32)
        + b1_ref[...], 0.0)
    z = jnp.dot(h, w2_ref[...], preferred_element_type=jnp.float32)
    zs_ref[0] = z[:, :H_OUT]
    zs_ref[1] = z[:, H_OUT:]


def _fin_body(acc_ref, zs_ref, dis_ref, b2_ref, out_ref):
    dis2 = _dis_col2(dis_ref[...])
    z = jnp.concatenate([zs_ref[0], zs_ref[1]], axis=1)
    t = jnp.concatenate([acc_ref[0], acc_ref[1]], axis=1)
    out_ref[...] = t + z * dis2 + b2_ref[...]


_mid_call = pl.pallas_call(
    _mid_body,
    grid=(NBB,),
    in_specs=[
        pl.BlockSpec((NC, RB, H_IN), lambda r: (0, r, 0)),
        pl.BlockSpec((RB, D_IN), lambda r: (r, 0)),
        pl.BlockSpec((1, RB), lambda r: (0, r)),
        pl.BlockSpec((D_IN, D_H), lambda r: (0, 0)),
        pl.BlockSpec((1, D_H), lambda r: (0, 0)),
        pl.BlockSpec((D_H, D_OUT), lambda r: (0, 0)),
    ],
    out_specs=pl.BlockSpec((NC, RB, H_OUT), lambda r: (0, r, 0)),
    out_shape=jax.ShapeDtypeStruct((NC, NPAD, H_OUT), jnp.float32),
)

_fin_call = pl.pallas_call(
    _fin_body,
    grid=(NBB,),
    in_specs=[
        pl.BlockSpec((NC, RB, H_OUT), lambda r: (0, r, 0)),
        pl.BlockSpec((NC, RB, H_OUT), lambda r: (0, r, 0)),
        pl.BlockSpec((1, RB), lambda r: (0, r)),
        pl.BlockSpec((1, D_OUT), lambda r: (0, 0)),
    ],
    out_specs=pl.BlockSpec((RB, D_OUT), lambda r: (r, 0)),
    out_shape=jax.ShapeDtypeStruct((NPAD, D_OUT), jnp.float32),
)


def kernel(x, edge_index, edge_weight, W1, b1, W2, b2):
    src = edge_index[0].astype(jnp.int32)
    dst = edge_index[1].astype(jnp.int32)
    ew = edge_weight.astype(jnp.float32)

    pad16 = EPAD16 - E
    src3 = jnp.concatenate(
        [src, jnp.zeros((pad16,), jnp.int32)]).reshape(NS, CH16, CHUNK)
    dst3 = jnp.concatenate(
        [dst, jnp.zeros((pad16,), jnp.int32)]).reshape(NS, CH16, CHUNK)
    ew16 = jnp.concatenate(
        [ew, jnp.zeros((pad16,), jnp.float32)]).reshape(NS, ESUB)

    x_p = jnp.concatenate([x, jnp.zeros((NPAD - N, D_IN), jnp.float32)])
    xsplit = jnp.stack([x_p[:, :H_IN], x_p[:, H_IN:]])
    zeros_h1 = jnp.zeros((ROWS_PER_SUB, H_IN), jnp.float32)
    zeros_h2 = jnp.zeros((ROWS_PER_SUB, H_OUT), jnp.float32)

    dis, warr = _norm_kernel(src3.reshape(NS, ESUB),
                             dst3.reshape(NS, ESUB), ew16)
    acc1 = _agg1(xsplit, src3, dst3, warr, zeros_h1)
    zss = _mid_call(acc1, x_p, dis, W1, b1.reshape(1, D_H), W2)

    acc2 = _agg2(zss, src3, dst3, warr, zeros_h2)
    out = _fin_call(acc2, zss, dis, b2.reshape(1, D_OUT))
    return out[:N]
